# baseline (speedup 1.0000x reference)
import functools

import jax
import jax.numpy as jnp
from jax import lax
from jax.experimental import pallas as pl
from jax.experimental.pallas import tpu as pltpu
from jax.experimental.pallas import tpu_sc as plsc

N = 10000
E = 320000
DIN = 128
DH = 64
DZ = 32

NC = 2
NS = 16
NW = NC * NS
C = 128
F = 4
K = 80
EW = K * C
E_PAD = NW * EW

N_PAD = 10240
NSLICE = N_PAD // NS

_MESH = dict(core_axis_name="c", subcore_axis_name="s", num_cores=NC,
             num_subcores=NS)
_SC_PARAMS = pltpu.CompilerParams(use_tc_tiling_on_sc=False,
                                  needs_layout_passes=False)


def _wids():
    cid = lax.axis_index("c")
    sid = lax.axis_index("s")
    return cid, sid, sid * NC + cid


def _deg_body(dstp, out, idx_v, ones_v, zb_v, shared):
    cid, sid, wid = _wids()
    for t in range(C // 16):
        ones_v[pl.ds(t * 16, 16)] = jnp.ones((16,), jnp.float32)
        zb_v[pl.ds(t * 16, 16)] = jnp.zeros((16,), jnp.float32)
    base = sid * NSLICE
    for t in range(NSLICE // C):
        pltpu.sync_copy(zb_v, shared.at[pl.ds(base + t * C, C)])
    pltpu.sync_copy(dstp.at[wid], idx_v)
    plsc.subcore_barrier()

    def step(j, carry):
        pltpu.sync_copy(ones_v, shared.at[idx_v.at[j]], add=True)
        return carry

    lax.fori_loop(0, K, step, 0)
    plsc.subcore_barrier()
    pltpu.sync_copy(shared.at[pl.ds(base, NSLICE)],
                    out.at[cid].at[pl.ds(base, NSLICE)])


_deg_call = functools.partial(
    pl.kernel,
    out_type=jax.ShapeDtypeStruct((NC, N_PAD), jnp.float32),
    mesh=plsc.VectorSubcoreMesh(**_MESH),
    compiler_params=_SC_PARAMS,
    scratch_types=[
        pltpu.VMEM((K, C), jnp.int32),
        pltpu.VMEM((C,), jnp.float32),
        pltpu.VMEM((C,), jnp.float32),
        pltpu.VMEM_SHARED((N_PAD,), jnp.float32),
    ],
)(_deg_body)


def _seg_body(table, srcp, dstp, out, idx_s, idx_d, rows0, rows1, shared,
              stab, gsem0, gsem1, ssem0, ssem1, *, nh):
    cid, sid, wid = _wids()
    base = sid * NSLICE
    pltpu.sync_copy(srcp.at[wid], idx_s)
    pltpu.sync_copy(dstp.at[wid], idx_d)

    def zrow(r, carry):
        for t in range(DZ // 16):
            rows0[r, pl.ds(t * 16, 16)] = jnp.zeros((16,), jnp.float32)
        return carry

    lax.fori_loop(0, C, zrow, 0)

    def gat4(s, rows, sem):
        for f in range(F):
            pltpu.async_copy(stab.at[idx_s.at[s * F + f]],
                             rows.at[pl.ds(f * C, C)], sem)

    def gat4_wait(s, rows, sem):
        for f in range(F):
            pltpu.make_async_copy(stab.at[idx_s.at[s * F + f]],
                                  rows.at[pl.ds(f * C, C)], sem).wait()

    def scat4(s, rows, sem):
        for f in range(F):
            pltpu.async_copy(rows.at[pl.ds(f * C, C)],
                             shared.at[idx_d.at[s * F + f]], sem, add=True)

    def scat4_wait(s, rows, sem):
        for f in range(F):
            pltpu.make_async_copy(rows.at[pl.ds(f * C, C)],
                                  shared.at[idx_d.at[s * F + f]], sem).wait()

    NSUP = K // F

    def step(ss, carry):
        s0 = 2 * ss
        s1 = s0 + 1
        gat4_wait(s0, rows0, gsem0)
        scat4(s0, rows0, ssem0)
        scat4_wait(s0, rows0, ssem0)

        @pl.when(s0 + 2 < NSUP)
        def _():
            gat4(s0 + 2, rows0, gsem0)

        gat4_wait(s1, rows1, gsem1)
        scat4(s1, rows1, ssem1)
        scat4_wait(s1, rows1, ssem1)

        @pl.when(s1 + 2 < NSUP)
        def _():
            gat4(s1 + 2, rows1, gsem1)

        return carry

    for h in range(nh):
        pltpu.sync_copy(table.at[h].at[pl.ds(base, NSLICE)],
                        stab.at[pl.ds(base, NSLICE)])
        for t in range(NSLICE // C):
            pltpu.sync_copy(rows0.at[pl.ds(0, C)],
                            shared.at[pl.ds(base + t * C, C)])
        plsc.subcore_barrier()
        gat4(0, rows0, gsem0)
        gat4(1, rows1, gsem1)
        lax.fori_loop(0, NSUP // 2, step, 0)
        plsc.subcore_barrier()
        for t in range(NSLICE // C):
            sl = pl.ds(base + t * C, C)
            pltpu.sync_copy(shared.at[sl], out.at[cid].at[h].at[sl])
        if h + 1 < nh:
            lax.fori_loop(0, C, zrow, 0)
            plsc.subcore_barrier()


def _make_seg_call(nh):
    scratch = [
        pltpu.VMEM((K, C), jnp.int32),
        pltpu.VMEM((K, C), jnp.int32),
        pltpu.VMEM((F * C, DZ), jnp.float32),
        pltpu.VMEM((F * C, DZ), jnp.float32),
        pltpu.VMEM_SHARED((N_PAD, DZ), jnp.float32),
        pltpu.VMEM_SHARED((N_PAD, DZ), jnp.float32),
    ]
    scratch += [pltpu.SemaphoreType.DMA] * 4
    return functools.partial(
        pl.kernel,
        out_type=jax.ShapeDtypeStruct((NC, nh, N_PAD, DZ), jnp.float32),
        mesh=plsc.VectorSubcoreMesh(**_MESH),
        compiler_params=_SC_PARAMS,
        scratch_types=scratch,
    )(functools.partial(_seg_body, nh=nh))


_seg_call_h = _make_seg_call(2)
_seg_call_z = _make_seg_call(1)


def _dec_body(ztab, srcp, dstp, out, idx_s, idx_d, zs0, zd0, zs1, zd1,
              obuf, stab, sem0, sem1):
    cid, sid, wid = _wids()
    base = sid * NSLICE
    pltpu.sync_copy(ztab.at[pl.ds(base, NSLICE)],
                    stab.at[pl.ds(base, NSLICE)])
    pltpu.sync_copy(srcp.at[wid], idx_s)
    pltpu.sync_copy(dstp.at[wid], idx_d)
    plsc.subcore_barrier()
    iota = lax.iota(jnp.int32, 16)

    def gat4(s, zs, zd, sem):
        for f in range(F):
            j = s * F + f
            pltpu.async_copy(stab.at[idx_s.at[j]],
                             zs.at[pl.ds(f * C, C)], sem)
            pltpu.async_copy(stab.at[idx_d.at[j]],
                             zd.at[pl.ds(f * C, C)], sem)

    def gat4_wait(s, zs, zd, sem):
        for f in range(F):
            j = s * F + f
            pltpu.make_async_copy(stab.at[idx_s.at[j]],
                                  zs.at[pl.ds(f * C, C)], sem).wait()
            pltpu.make_async_copy(stab.at[idx_d.at[j]],
                                  zd.at[pl.ds(f * C, C)], sem).wait()

    def compute4(s, zs, zd):
        for f in range(F):
            j = s * F + f

            def group(g, carry, f=f, j=j):
                rows = f * C + g * 16 + iota
                accs = [jnp.zeros((16,), jnp.float32) for _ in range(4)]
                for cc in range(DZ):
                    col = jnp.full((16,), cc, jnp.int32)
                    accs[cc % 4] = accs[cc % 4] + (
                        plsc.load_gather(zs, [rows, col])
                        * plsc.load_gather(zd, [rows, col]))
                obuf[j, pl.ds(g * 16, 16)] = (
                    (accs[0] + accs[1]) + (accs[2] + accs[3]))
                return carry

            lax.fori_loop(0, C // 16, group, 0)

    NSUP = K // F
    gat4(0, zs0, zd0, sem0)
    gat4(1, zs1, zd1, sem1)

    def step(ss, carry):
        s0 = 2 * ss
        s1 = s0 + 1
        gat4_wait(s0, zs0, zd0, sem0)
        compute4(s0, zs0, zd0)

        @pl.when(s0 + 2 < NSUP)
        def _():
            gat4(s0 + 2, zs0, zd0, sem0)

        gat4_wait(s1, zs1, zd1, sem1)
        compute4(s1, zs1, zd1)

        @pl.when(s1 + 2 < NSUP)
        def _():
            gat4(s1 + 2, zs1, zd1, sem1)

        return carry

    lax.fori_loop(0, NSUP // 2, step, 0)
    pltpu.sync_copy(obuf, out.at[wid])


_dec_call = functools.partial(
    pl.kernel,
    out_type=jax.ShapeDtypeStruct((NW, K, C), jnp.float32),
    mesh=plsc.VectorSubcoreMesh(**_MESH),
    compiler_params=_SC_PARAMS,
    scratch_types=[
        pltpu.VMEM((K, C), jnp.int32),
        pltpu.VMEM((K, C), jnp.int32),
        pltpu.VMEM((F * C, DZ), jnp.float32),
        pltpu.VMEM((F * C, DZ), jnp.float32),
        pltpu.VMEM((F * C, DZ), jnp.float32),
        pltpu.VMEM((F * C, DZ), jnp.float32),
        pltpu.VMEM((K, C), jnp.float32),
        pltpu.VMEM_SHARED((N_PAD, DZ), jnp.float32),
        pltpu.SemaphoreType.DMA,
        pltpu.SemaphoreType.DMA,
    ],
)(_dec_body)


_B = 512
_GRID = N_PAD // _B


def _dinv_of(degt):
    return lax.rsqrt(degt[:, 0:1] + degt[:, 1:2] + 1.0)


def _tc1_body(degt_ref, x_ref, w1_ref, o_ref):
    dinv = _dinv_of(degt_ref[...])
    h = jnp.dot(x_ref[...], w1_ref[...], preferred_element_type=jnp.float32)
    hs = h * dinv
    o_ref[0] = hs[:, :DZ]
    o_ref[1] = hs[:, DZ:]


def _tc1(degt, x_pad, w1):
    return pl.pallas_call(
        _tc1_body,
        grid=(_GRID,),
        in_specs=[
            pl.BlockSpec((_B, NC), lambda i: (i, 0)),
            pl.BlockSpec((_B, DIN), lambda i: (i, 0)),
            pl.BlockSpec((DIN, DH), lambda i: (0, 0)),
        ],
        out_specs=pl.BlockSpec((2, _B, DZ), lambda i: (0, i, 0)),
        out_shape=jax.ShapeDtypeStruct((2, N_PAD, DZ), jnp.float32),
    )(degt, x_pad, w1)


def _tc2_body(degt_ref, p_ref, hs1_ref, b1_ref, w2_ref, o_ref):
    dinv = _dinv_of(degt_ref[...])
    agg_a = p_ref[0, 0] + p_ref[1, 0] + hs1_ref[0]
    agg_b = p_ref[0, 1] + p_ref[1, 1] + hs1_ref[1]
    agg = jnp.concatenate([agg_a, agg_b], axis=1)
    a1 = jnp.maximum(dinv * agg + b1_ref[...], 0.0)
    h2 = jnp.dot(a1, w2_ref[...], preferred_element_type=jnp.float32)
    o_ref[...] = h2 * dinv


def _tc2(degt, parts1, hs1, b1, w2):
    return pl.pallas_call(
        _tc2_body,
        grid=(_GRID,),
        in_specs=[
            pl.BlockSpec((_B, NC), lambda i: (i, 0)),
            pl.BlockSpec((NC, 2, _B, DZ), lambda i: (0, 0, i, 0)),
            pl.BlockSpec((2, _B, DZ), lambda i: (0, i, 0)),
            pl.BlockSpec((1, DH), lambda i: (0, 0)),
            pl.BlockSpec((DH, DZ), lambda i: (0, 0)),
        ],
        out_specs=pl.BlockSpec((_B, DZ), lambda i: (i, 0)),
        out_shape=jax.ShapeDtypeStruct((N_PAD, DZ), jnp.float32),
    )(degt, parts1, hs1, b1, w2)


def _tc3_body(degt_ref, p_ref, hs2_ref, b2_ref, o_ref):
    dinv = _dinv_of(degt_ref[...])
    agg = p_ref[0, 0] + p_ref[1, 0] + hs2_ref[...]
    o_ref[...] = dinv * agg + b2_ref[...]


def _tc3(degt, parts2, hs2, b2):
    return pl.pallas_call(
        _tc3_body,
        grid=(_GRID,),
        in_specs=[
            pl.BlockSpec((_B, NC), lambda i: (i, 0)),
            pl.BlockSpec((NC, 1, _B, DZ), lambda i: (0, 0, i, 0)),
            pl.BlockSpec((_B, DZ), lambda i: (i, 0)),
            pl.BlockSpec((1, DZ), lambda i: (0, 0)),
        ],
        out_specs=pl.BlockSpec((_B, DZ), lambda i: (i, 0)),
        out_shape=jax.ShapeDtypeStruct((N_PAD, DZ), jnp.float32),
    )(degt, parts2, hs2, b2)


@jax.jit
def kernel(x, edge_index, W1, b1, W2, b2):
    src = edge_index[0]
    dst = edge_index[1]
    pad = jnp.full((E_PAD - E,), N, jnp.int32)
    srcp = jnp.concatenate([src, pad]).reshape(NW, K, C)
    dstp = jnp.concatenate([dst, pad]).reshape(NW, K, C)
    x_pad = jnp.pad(x, ((0, N_PAD - N), (0, 0)))

    deg_parts = _deg_call(dstp)
    degt = deg_parts.T

    hs1 = _tc1(degt, x_pad, W1)
    parts1 = _seg_call_h(hs1, srcp, dstp)
    hs2 = _tc2(degt, parts1, hs1, b1.reshape(1, DH), W2)
    parts2 = _seg_call_z(hs2.reshape(1, N_PAD, DZ), srcp, dstp)
    z = _tc3(degt, parts2, hs2, b2.reshape(1, DZ))

    recon = _dec_call(z, srcp, dstp).reshape(-1)[:E]
    return z[:N], recon


# speedup vs baseline: 1.5537x; 1.5537x over previous
import functools

import jax
import jax.numpy as jnp
from jax import lax
from jax.experimental import pallas as pl
from jax.experimental.pallas import tpu as pltpu
from jax.experimental.pallas import tpu_sc as plsc

N = 10000
E = 320000
DIN = 128
DH = 64
DZ = 32

NC = 2
NS = 16
NW = NC * NS
C = 128
F = 4
K = 80
EW = K * C
E_PAD = NW * EW

N_PAD = 10240
NSLICE = N_PAD // NS

_MESH = dict(core_axis_name="c", subcore_axis_name="s", num_cores=NC,
             num_subcores=NS)
_SC_PARAMS = pltpu.CompilerParams(use_tc_tiling_on_sc=False,
                                  needs_layout_passes=False)


def _wids():
    cid = lax.axis_index("c")
    sid = lax.axis_index("s")
    return cid, sid, sid * NC + cid


def _deg_body(dstp, out, idx_v, ones_v, zb_v, shared):
    cid, sid, wid = _wids()
    for t in range(C // 16):
        ones_v[pl.ds(t * 16, 16)] = jnp.ones((16,), jnp.float32)
        zb_v[pl.ds(t * 16, 16)] = jnp.zeros((16,), jnp.float32)
    base = sid * NSLICE
    for t in range(NSLICE // C):
        pltpu.sync_copy(zb_v, shared.at[pl.ds(base + t * C, C)])
    pltpu.sync_copy(dstp.at[wid], idx_v)
    plsc.subcore_barrier()

    def step(j, carry):
        pltpu.sync_copy(ones_v, shared.at[idx_v.at[j]], add=True)
        return carry

    lax.fori_loop(0, K, step, 0)
    plsc.subcore_barrier()
    pltpu.sync_copy(shared.at[pl.ds(base, NSLICE)],
                    out.at[cid].at[pl.ds(base, NSLICE)])


_deg_call = functools.partial(
    pl.kernel,
    out_type=jax.ShapeDtypeStruct((NC, N_PAD), jnp.float32),
    mesh=plsc.VectorSubcoreMesh(**_MESH),
    compiler_params=_SC_PARAMS,
    scratch_types=[
        pltpu.VMEM((K, C), jnp.int32),
        pltpu.VMEM((C,), jnp.float32),
        pltpu.VMEM((C,), jnp.float32),
        pltpu.VMEM_SHARED((N_PAD,), jnp.float32),
    ],
)(_deg_body)


def _seg_body(table, srcp, dstp, out, idx_s, idx_d, rows0, rows1, shared,
              stab, gsem0, gsem1, ssem0, ssem1, *, nh):
    cid, sid, wid = _wids()
    base = sid * NSLICE
    pltpu.sync_copy(srcp.at[wid], idx_s)
    pltpu.sync_copy(dstp.at[wid], idx_d)

    def zrow(r, carry):
        for t in range(DZ // 16):
            rows0[r, pl.ds(t * 16, 16)] = jnp.zeros((16,), jnp.float32)
        return carry

    lax.fori_loop(0, C, zrow, 0)

    def gat4(s, rows, sem):
        for f in range(F):
            pltpu.async_copy(stab.at[idx_s.at[s * F + f]],
                             rows.at[pl.ds(f * C, C)], sem)

    def gat4_wait(s, rows, sem):
        for f in range(F):
            pltpu.make_async_copy(stab.at[idx_s.at[s * F + f]],
                                  rows.at[pl.ds(f * C, C)], sem).wait()

    def scat4(s, rows, sem):
        for f in range(F):
            pltpu.async_copy(rows.at[pl.ds(f * C, C)],
                             shared.at[idx_d.at[s * F + f]], sem, add=True)

    def scat4_wait(s, rows, sem):
        for f in range(F):
            pltpu.make_async_copy(rows.at[pl.ds(f * C, C)],
                                  shared.at[idx_d.at[s * F + f]], sem).wait()

    NSUP = K // F

    def step(ss, carry):
        s0 = 2 * ss
        s1 = s0 + 1
        gat4_wait(s0, rows0, gsem0)
        scat4(s0, rows0, ssem0)
        scat4_wait(s0, rows0, ssem0)

        @pl.when(s0 + 2 < NSUP)
        def _():
            gat4(s0 + 2, rows0, gsem0)

        gat4_wait(s1, rows1, gsem1)
        scat4(s1, rows1, ssem1)
        scat4_wait(s1, rows1, ssem1)

        @pl.when(s1 + 2 < NSUP)
        def _():
            gat4(s1 + 2, rows1, gsem1)

        return carry

    for h in range(nh):
        pltpu.sync_copy(table.at[h].at[pl.ds(base, NSLICE)],
                        stab.at[pl.ds(base, NSLICE)])
        for t in range(NSLICE // C):
            pltpu.sync_copy(rows0.at[pl.ds(0, C)],
                            shared.at[pl.ds(base + t * C, C)])
        plsc.subcore_barrier()
        gat4(0, rows0, gsem0)
        gat4(1, rows1, gsem1)
        lax.fori_loop(0, NSUP // 2, step, 0)
        plsc.subcore_barrier()
        for t in range(NSLICE // C):
            sl = pl.ds(base + t * C, C)
            pltpu.sync_copy(shared.at[sl], out.at[cid].at[h].at[sl])
        if h + 1 < nh:
            lax.fori_loop(0, C, zrow, 0)
            plsc.subcore_barrier()


def _make_seg_call(nh):
    scratch = [
        pltpu.VMEM((K, C), jnp.int32),
        pltpu.VMEM((K, C), jnp.int32),
        pltpu.VMEM((F * C, DZ), jnp.float32),
        pltpu.VMEM((F * C, DZ), jnp.float32),
        pltpu.VMEM_SHARED((N_PAD, DZ), jnp.float32),
        pltpu.VMEM_SHARED((N_PAD, DZ), jnp.float32),
    ]
    scratch += [pltpu.SemaphoreType.DMA] * 4
    return functools.partial(
        pl.kernel,
        out_type=jax.ShapeDtypeStruct((NC, nh, N_PAD, DZ), jnp.float32),
        mesh=plsc.VectorSubcoreMesh(**_MESH),
        compiler_params=_SC_PARAMS,
        scratch_types=scratch,
    )(functools.partial(_seg_body, nh=nh))


_seg_call_h = _make_seg_call(2)
_seg_call_z = _make_seg_call(1)


def _dec_body(ztab, srcp, dstp, out, idx_s, idx_d, zs0, zd0, zs1, zd1, fb0,
              fb1, obuf, stab, sem0, sem1):
    cid, sid, wid = _wids()
    base = sid * NSLICE
    pltpu.sync_copy(ztab.at[pl.ds(base, NSLICE)],
                    stab.at[pl.ds(base, NSLICE)])
    pltpu.sync_copy(srcp.at[wid], idx_s)
    pltpu.sync_copy(dstp.at[wid], idx_d)
    plsc.subcore_barrier()
    iota = lax.iota(jnp.int32, 16)

    def gat4(s, zs, zd, sem):
        for f in range(F):
            j = s * F + f
            pltpu.async_copy(stab.at[idx_s.at[j]],
                             zs.at[pl.ds(f * C, C)], sem)
            pltpu.async_copy(stab.at[idx_d.at[j]],
                             zd.at[pl.ds(f * C, C)], sem)

    def gat4_wait(s, zs, zd, sem):
        for f in range(F):
            j = s * F + f
            pltpu.make_async_copy(stab.at[idx_s.at[j]],
                                  zs.at[pl.ds(f * C, C)], sem).wait()
            pltpu.make_async_copy(stab.at[idx_d.at[j]],
                                  zd.at[pl.ds(f * C, C)], sem).wait()

    def compute4(s, zs, zd):
        def one_group(f, j, g, fb):
            for r in range(16):
                e = f * C + g * 16 + r
                fb[r] = (zs[e, pl.ds(0, 16)] * zd[e, pl.ds(0, 16)]
                         + zs[e, pl.ds(16, 16)] * zd[e, pl.ds(16, 16)])
            accs = [plsc.load_gather(fb, [iota, jnp.full((16,), cc,
                                                          jnp.int32)])
                    for cc in range(4)]
            for cc in range(4, 16):
                accs[cc % 4] = accs[cc % 4] + plsc.load_gather(
                    fb, [iota, jnp.full((16,), cc, jnp.int32)])
            obuf[j, pl.ds(g * 16, 16)] = ((accs[0] + accs[1])
                                          + (accs[2] + accs[3]))

        for f in range(F):
            j = s * F + f

            def gpair(gg, carry, f=f, j=j):
                one_group(f, j, 2 * gg, fb0)
                one_group(f, j, 2 * gg + 1, fb1)
                return carry

            lax.fori_loop(0, C // 32, gpair, 0)

    NSUP = K // F
    gat4(0, zs0, zd0, sem0)
    gat4(1, zs1, zd1, sem1)

    def step(ss, carry):
        s0 = 2 * ss
        s1 = s0 + 1
        gat4_wait(s0, zs0, zd0, sem0)
        compute4(s0, zs0, zd0)

        @pl.when(s0 + 2 < NSUP)
        def _():
            gat4(s0 + 2, zs0, zd0, sem0)

        gat4_wait(s1, zs1, zd1, sem1)
        compute4(s1, zs1, zd1)

        @pl.when(s1 + 2 < NSUP)
        def _():
            gat4(s1 + 2, zs1, zd1, sem1)

        return carry

    lax.fori_loop(0, NSUP // 2, step, 0)
    pltpu.sync_copy(obuf, out.at[wid])


_dec_call = functools.partial(
    pl.kernel,
    out_type=jax.ShapeDtypeStruct((NW, K, C), jnp.float32),
    mesh=plsc.VectorSubcoreMesh(**_MESH),
    compiler_params=_SC_PARAMS,
    scratch_types=[
        pltpu.VMEM((K, C), jnp.int32),
        pltpu.VMEM((K, C), jnp.int32),
        pltpu.VMEM((F * C, DZ), jnp.float32),
        pltpu.VMEM((F * C, DZ), jnp.float32),
        pltpu.VMEM((F * C, DZ), jnp.float32),
        pltpu.VMEM((F * C, DZ), jnp.float32),
        pltpu.VMEM((16, 16), jnp.float32),
        pltpu.VMEM((16, 16), jnp.float32),
        pltpu.VMEM((K, C), jnp.float32),
        pltpu.VMEM_SHARED((N_PAD, DZ), jnp.float32),
        pltpu.SemaphoreType.DMA,
        pltpu.SemaphoreType.DMA,
    ],
)(_dec_body)


_B = 512
_GRID = N_PAD // _B


def _dinv_of(degt):
    return lax.rsqrt(degt[:, 0:1] + degt[:, 1:2] + 1.0)


def _tc1_body(degt_ref, x_ref, w1_ref, o_ref):
    dinv = _dinv_of(degt_ref[...])
    h = jnp.dot(x_ref[...], w1_ref[...], preferred_element_type=jnp.float32)
    hs = h * dinv
    o_ref[0] = hs[:, :DZ]
    o_ref[1] = hs[:, DZ:]


def _tc1(degt, x_pad, w1):
    return pl.pallas_call(
        _tc1_body,
        grid=(_GRID,),
        in_specs=[
            pl.BlockSpec((_B, NC), lambda i: (i, 0)),
            pl.BlockSpec((_B, DIN), lambda i: (i, 0)),
            pl.BlockSpec((DIN, DH), lambda i: (0, 0)),
        ],
        out_specs=pl.BlockSpec((2, _B, DZ), lambda i: (0, i, 0)),
        out_shape=jax.ShapeDtypeStruct((2, N_PAD, DZ), jnp.float32),
    )(degt, x_pad, w1)


def _tc2_body(degt_ref, p_ref, hs1_ref, b1_ref, w2_ref, o_ref):
    dinv = _dinv_of(degt_ref[...])
    agg_a = p_ref[0, 0] + p_ref[1, 0] + hs1_ref[0]
    agg_b = p_ref[0, 1] + p_ref[1, 1] + hs1_ref[1]
    agg = jnp.concatenate([agg_a, agg_b], axis=1)
    a1 = jnp.maximum(dinv * agg + b1_ref[...], 0.0)
    h2 = jnp.dot(a1, w2_ref[...], preferred_element_type=jnp.float32)
    o_ref[...] = h2 * dinv


def _tc2(degt, parts1, hs1, b1, w2):
    return pl.pallas_call(
        _tc2_body,
        grid=(_GRID,),
        in_specs=[
            pl.BlockSpec((_B, NC), lambda i: (i, 0)),
            pl.BlockSpec((NC, 2, _B, DZ), lambda i: (0, 0, i, 0)),
            pl.BlockSpec((2, _B, DZ), lambda i: (0, i, 0)),
            pl.BlockSpec((1, DH), lambda i: (0, 0)),
            pl.BlockSpec((DH, DZ), lambda i: (0, 0)),
        ],
        out_specs=pl.BlockSpec((_B, DZ), lambda i: (i, 0)),
        out_shape=jax.ShapeDtypeStruct((N_PAD, DZ), jnp.float32),
    )(degt, parts1, hs1, b1, w2)


def _tc3_body(degt_ref, p_ref, hs2_ref, b2_ref, o_ref):
    dinv = _dinv_of(degt_ref[...])
    agg = p_ref[0, 0] + p_ref[1, 0] + hs2_ref[...]
    o_ref[...] = dinv * agg + b2_ref[...]


def _tc3(degt, parts2, hs2, b2):
    return pl.pallas_call(
        _tc3_body,
        grid=(_GRID,),
        in_specs=[
            pl.BlockSpec((_B, NC), lambda i: (i, 0)),
            pl.BlockSpec((NC, 1, _B, DZ), lambda i: (0, 0, i, 0)),
            pl.BlockSpec((_B, DZ), lambda i: (i, 0)),
            pl.BlockSpec((1, DZ), lambda i: (0, 0)),
        ],
        out_specs=pl.BlockSpec((_B, DZ), lambda i: (i, 0)),
        out_shape=jax.ShapeDtypeStruct((N_PAD, DZ), jnp.float32),
    )(degt, parts2, hs2, b2)


@jax.jit
def kernel(x, edge_index, W1, b1, W2, b2):
    src = edge_index[0]
    dst = edge_index[1]
    pad = jnp.full((E_PAD - E,), N, jnp.int32)
    srcp = jnp.concatenate([src, pad]).reshape(NW, K, C)
    dstp = jnp.concatenate([dst, pad]).reshape(NW, K, C)
    x_pad = jnp.pad(x, ((0, N_PAD - N), (0, 0)))

    deg_parts = _deg_call(dstp)
    degt = deg_parts.T

    hs1 = _tc1(degt, x_pad, W1)
    parts1 = _seg_call_h(hs1, srcp, dstp)
    hs2 = _tc2(degt, parts1, hs1, b1.reshape(1, DH), W2)
    parts2 = _seg_call_z(hs2.reshape(1, N_PAD, DZ), srcp, dstp)
    z = _tc3(degt, parts2, hs2, b2.reshape(1, DZ))

    recon = _dec_call(z, srcp, dstp).reshape(-1)[:E]
    return z[:N], recon
